# Initial kernel scaffold; baseline (speedup 1.0000x reference)
import jax
import jax.numpy as jnp
from jax.experimental import pallas as pl


def kernel(x, pe):
    raise NotImplementedError("write your pallas kernel here")


# speedup vs baseline: 3.5020x; 3.5020x over previous
import functools

import jax
import jax.numpy as jnp
from jax import lax
from jax.experimental import pallas as pl
from jax.experimental.pallas import tpu as pltpu
from jax.experimental.pallas import tpu_sc as plsc

D_MODEL = 128
CHUNK = 128


def _make_sc_gather(n_rows: int, nw: int, nc: int):
    rows_per_w = n_rows // nw
    n_chunks = rows_per_w // CHUNK
    mesh = plsc.VectorSubcoreMesh(core_axis_name="c", subcore_axis_name="s")

    @functools.partial(
        pl.kernel,
        mesh=mesh,
        out_type=jax.ShapeDtypeStruct((n_rows, D_MODEL), jnp.float32),
        scratch_types=[
            pltpu.VMEM((n_chunks, CHUNK), jnp.int32),
            pltpu.VMEM((CHUNK, D_MODEL), jnp.float32),
            pltpu.SemaphoreType.DMA,
        ],
    )
    def k(idx_hbm, pe_hbm, out_hbm, idx_v, rows_v, sem):
        wid = lax.axis_index("s") * nc + lax.axis_index("c")
        base = wid * rows_per_w
        pltpu.sync_copy(idx_hbm.at[wid], idx_v)

        def chunk(j, _):
            pltpu.async_copy(pe_hbm.at[idx_v.at[j]], rows_v, sem).wait()
            pltpu.sync_copy(rows_v, out_hbm.at[pl.ds(base + j * CHUNK, CHUNK)])
            return _

        lax.fori_loop(0, n_chunks, chunk, None)

    return k


def kernel(x, pe):
    b, s = x.shape
    n_rows = b * s
    info = plsc.get_sparse_core_info()
    nc, ns = info.num_cores, info.num_subcores
    nw = nc * ns
    rows_per_w = n_rows // nw
    idx = x.reshape(nw, rows_per_w // CHUNK, CHUNK).astype(jnp.int32)
    out = _make_sc_gather(n_rows, nw, nc)(idx, pe)
    return out.reshape(b, s, D_MODEL)


# speedup vs baseline: 3.5947x; 1.0265x over previous
import functools

import jax
import jax.numpy as jnp
from jax import lax
from jax.experimental import pallas as pl
from jax.experimental.pallas import tpu as pltpu
from jax.experimental.pallas import tpu_sc as plsc

D_MODEL = 128
CHUNK = 64
K = 4


def _make_sc_gather(n_rows: int, nw: int, nc: int):
    rows_per_w = n_rows // nw
    n_chunks = rows_per_w // CHUNK
    n_pairs = n_chunks // (2 * K)
    mesh = plsc.VectorSubcoreMesh(core_axis_name="c", subcore_axis_name="s")

    @functools.partial(
        pl.kernel,
        mesh=mesh,
        out_type=jax.ShapeDtypeStruct((n_rows, D_MODEL), jnp.float32),
        scratch_types=[
            pltpu.VMEM((n_chunks, CHUNK), jnp.int32),
            pltpu.VMEM((2 * K, CHUNK, D_MODEL), jnp.float32),
            pltpu.SemaphoreType.DMA,
            pltpu.SemaphoreType.DMA,
            pltpu.SemaphoreType.DMA,
            pltpu.SemaphoreType.DMA,
        ],
    )
    def k(idx_hbm, pe_hbm, out_hbm, idx_v, rows_v, gsem_a, gsem_b, ssem_a, ssem_b):
        wid = lax.axis_index("s") * nc + lax.axis_index("c")
        base = wid * rows_per_w
        pltpu.sync_copy(idx_hbm.at[wid], idx_v)

        def start_g(j, b, sem):
            pltpu.async_copy(pe_hbm.at[idx_v.at[j]], rows_v.at[b], sem)

        def drain_g(b, sem):
            pltpu.make_async_copy(pe_hbm.at[idx_v.at[0]], rows_v.at[b], sem).wait()

        def start_s(j, b, sem):
            pltpu.async_copy(rows_v.at[b], out_hbm.at[pl.ds(base + j * CHUNK, CHUNK)], sem)

        def drain_s(b, sem):
            pltpu.make_async_copy(rows_v.at[b], out_hbm.at[pl.ds(base, CHUNK)], sem).wait()

        for b in range(K):
            start_g(b, b, gsem_a)
        for b in range(K):
            start_g(K + b, K + b, gsem_b)

        def body(p, _):
            c0 = p * 2 * K
            for b in range(K):
                drain_g(b, gsem_a)
            for b in range(K):
                start_s(c0 + b, b, ssem_a)
            for b in range(K):
                drain_g(K + b, gsem_b)
            for b in range(K):
                start_s(c0 + K + b, K + b, ssem_b)
            for b in range(K):
                drain_s(b, ssem_a)
            for b in range(K):
                start_g(c0 + 2 * K + b, b, gsem_a)
            for b in range(K):
                drain_s(K + b, ssem_b)
            for b in range(K):
                start_g(c0 + 3 * K + b, K + b, gsem_b)
            return _

        lax.fori_loop(0, n_pairs - 1, body, None)

        cl = (n_pairs - 1) * 2 * K
        for b in range(K):
            drain_g(b, gsem_a)
        for b in range(K):
            start_s(cl + b, b, ssem_a)
        for b in range(K):
            drain_g(K + b, gsem_b)
        for b in range(K):
            start_s(cl + K + b, K + b, ssem_b)
        for b in range(K):
            drain_s(b, ssem_a)
        for b in range(K):
            drain_s(K + b, ssem_b)

    return k


def kernel(x, pe):
    b, s = x.shape
    n_rows = b * s
    info = plsc.get_sparse_core_info()
    nc, ns = info.num_cores, info.num_subcores
    nw = nc * ns
    rows_per_w = n_rows // nw
    idx = x.reshape(nw, rows_per_w // CHUNK, CHUNK).astype(jnp.int32)
    out = _make_sc_gather(n_rows, nw, nc)(idx, pe)
    return out.reshape(b, s, D_MODEL)


# speedup vs baseline: 10.8053x; 3.0059x over previous
import functools

import jax
import jax.numpy as jnp
from jax import lax
from jax.experimental import pallas as pl
from jax.experimental.pallas import tpu as pltpu
from jax.experimental.pallas import tpu_sc as plsc

D_MODEL = 128
CHUNK = 64
K = 4


def _make_sc_gather(n_rows: int, nw: int, nc: int):
    rows_per_w = n_rows // nw
    n_chunks = rows_per_w // CHUNK
    n_pairs = n_chunks // (2 * K)
    mesh = plsc.VectorSubcoreMesh(core_axis_name="c", subcore_axis_name="s")

    @functools.partial(
        pl.kernel,
        mesh=mesh,
        out_type=jax.ShapeDtypeStruct((n_rows, D_MODEL), jnp.float32),
        scratch_types=[
            pltpu.VMEM((n_chunks, CHUNK), jnp.int32),
            pltpu.VMEM((2 * K, CHUNK, D_MODEL), jnp.float32),
            pltpu.VMEM_SHARED((200, D_MODEL), jnp.float32),
            pltpu.SemaphoreType.DMA,
            pltpu.SemaphoreType.DMA,
            pltpu.SemaphoreType.DMA,
            pltpu.SemaphoreType.DMA,
        ],
    )
    def k(idx_hbm, pe_hbm, out_hbm, idx_v, rows_v, pe_sh, gsem_a, gsem_b, ssem_a, ssem_b):
        sid = lax.axis_index("s")
        wid = sid * nc + lax.axis_index("c")
        base = wid * rows_per_w
        @pl.when(sid == 0)
        def _():
            pltpu.sync_copy(pe_hbm, pe_sh)

        pltpu.sync_copy(idx_hbm.at[wid], idx_v)
        plsc.subcore_barrier()

        def start_g(j, b, sem):
            pltpu.async_copy(pe_sh.at[idx_v.at[j]], rows_v.at[b], sem)

        def drain_g(b, sem):
            pltpu.make_async_copy(pe_sh.at[idx_v.at[0]], rows_v.at[b], sem).wait()

        def start_s(j, b, sem):
            pltpu.async_copy(rows_v.at[b], out_hbm.at[pl.ds(base + j * CHUNK, CHUNK)], sem)

        def drain_s(b, sem):
            pltpu.make_async_copy(rows_v.at[b], out_hbm.at[pl.ds(base, CHUNK)], sem).wait()

        for b in range(K):
            start_g(b, b, gsem_a)
        for b in range(K):
            start_g(K + b, K + b, gsem_b)

        def body(p, _):
            c0 = p * 2 * K
            for b in range(K):
                drain_g(b, gsem_a)
            for b in range(K):
                start_s(c0 + b, b, ssem_a)
            for b in range(K):
                drain_g(K + b, gsem_b)
            for b in range(K):
                start_s(c0 + K + b, K + b, ssem_b)
            for b in range(K):
                drain_s(b, ssem_a)
            for b in range(K):
                start_g(c0 + 2 * K + b, b, gsem_a)
            for b in range(K):
                drain_s(K + b, ssem_b)
            for b in range(K):
                start_g(c0 + 3 * K + b, K + b, gsem_b)
            return _

        lax.fori_loop(0, n_pairs - 1, body, None)

        cl = (n_pairs - 1) * 2 * K
        for b in range(K):
            drain_g(b, gsem_a)
        for b in range(K):
            start_s(cl + b, b, ssem_a)
        for b in range(K):
            drain_g(K + b, gsem_b)
        for b in range(K):
            start_s(cl + K + b, K + b, ssem_b)
        for b in range(K):
            drain_s(b, ssem_a)
        for b in range(K):
            drain_s(K + b, ssem_b)

    return k


def kernel(x, pe):
    b, s = x.shape
    n_rows = b * s
    info = plsc.get_sparse_core_info()
    nc, ns = info.num_cores, info.num_subcores
    nw = nc * ns
    rows_per_w = n_rows // nw
    idx = x.reshape(nw, rows_per_w // CHUNK, CHUNK).astype(jnp.int32)
    out = _make_sc_gather(n_rows, nw, nc)(idx, pe)
    return out.reshape(b, s, D_MODEL)
